# baseline (speedup 1.0000x reference)
import functools

import jax
import jax.numpy as jnp
from jax import lax
from jax.experimental import pallas as pl
from jax.experimental.pallas import tpu as pltpu
from jax.experimental.pallas import tpu_sc as plsc

_N, _D, _K = 16384, 256, 1024
_BN = 256
_NB = _N // _BN
_NC, _NS = 2, 16
_NW = _NC * _NS
_BPW = _N // _NW
_CH = 128
_NCH = _BPW // _CH


def _rne_bf16(x):
    u = lax.bitcast_convert_type(x, jnp.int32)
    r = (u + jnp.int32(0x7FFF) + ((u >> 16) & jnp.int32(1))) & jnp.int32(-65536)
    return lax.bitcast_convert_type(r, jnp.float32)


def _chain_residual(z, qs):
    r = z
    cs = []
    for q in qs:
        qr = _rne_bf16(q)
        c = r + (qr - r)
        cs.append(c)
        r = r - c
    return r, cs


def _tc_stage_body(nprev, *refs):
    z_ref = refs[0]
    q_refs = refs[1:1 + nprev]
    w_ref = refs[1 + nprev]
    idx_ref = refs[2 + nprev]
    loss_ref = refs[3 + nprev]
    wb_scr = refs[4 + nprev]
    c2_scr = refs[5 + nprev]

    i = pl.program_id(0)

    @pl.when(i == 0)
    def _():
        w = w_ref[...]
        wb_scr[...] = w.astype(jnp.bfloat16)
        c2 = jnp.sum(w * w, axis=1)
        c2_scr[...] = jnp.broadcast_to(c2[None, :], (8, _K))

    z = z_ref[...]
    r, _ = _chain_residual(z, [q[...] for q in q_refs])

    cl2 = lax.dot_general(
        (r + r).astype(jnp.bfloat16), wb_scr[...],
        (((1,), (1,)), ((), ())), preferred_element_type=jnp.float32)
    l2 = jnp.sum(r * r, axis=1, keepdims=True)
    l2b = jnp.broadcast_to(l2, (_BN, 128))
    c2 = c2_scr[0:1, :]

    m_run = (l2b - cl2[:, 0:128]) + jnp.broadcast_to(c2[:, 0:128], (_BN, 128))
    i_run = jnp.zeros((_BN, 128), jnp.int32)
    for c in range(1, _K // 128):
        sl = slice(c * 128, (c + 1) * 128)
        dc = (l2b - cl2[:, sl]) + jnp.broadcast_to(c2[:, sl], (_BN, 128))
        lt = dc < m_run
        m_run = jnp.where(lt, dc, m_run)
        i_run = jnp.where(lt, jnp.int32(c), i_run)

    m = jnp.min(m_run, axis=1, keepdims=True)
    lane = lax.broadcasted_iota(jnp.int32, (_BN, 128), 1)
    kk = i_run * jnp.int32(128) + lane
    idx = jnp.min(jnp.where(m_run == m, kk, jnp.int32(_K)), axis=1)
    idx_ref[...] = idx.reshape(1, 1, _BN)

    rows8 = lax.broadcasted_iota(jnp.int32, (8, 128), 0)
    cols8 = lax.broadcasted_iota(jnp.int32, (8, 128), 1)
    part = jnp.where((rows8 == 0) & (cols8 == 0), jnp.sum(m), 0.0)

    @pl.when(i == 0)
    def _():
        loss_ref[...] = jnp.zeros_like(loss_ref)

    loss_ref[...] += part


def _tc_stage(z, qs, w):
    nprev = len(qs)
    grid = (_NB,)
    row_spec = pl.BlockSpec((_BN, _D), lambda i: (i, 0))
    in_specs = ([row_spec] + [row_spec] * nprev
                + [pl.BlockSpec((_K, _D), lambda i: (0, 0))])
    out_specs = [
        pl.BlockSpec((1, 1, _BN), lambda i: (i, 0, 0)),
        pl.BlockSpec((8, 128), lambda i: (0, 0)),
    ]
    out_shape = [
        jax.ShapeDtypeStruct((_NB, 1, _BN), jnp.int32),
        jax.ShapeDtypeStruct((8, 128), jnp.float32),
    ]
    idx, loss = pl.pallas_call(
        functools.partial(_tc_stage_body, nprev),
        grid=grid,
        in_specs=in_specs,
        out_specs=out_specs,
        out_shape=out_shape,
        scratch_shapes=[
            pltpu.VMEM((_K, _D), jnp.bfloat16),
            pltpu.VMEM((8, _K), jnp.float32),
        ],
    )(z, *qs, w)
    return idx.reshape(_N), loss[0, 0]


def _tc_final_body(z_ref, q1_ref, q2_ref, q3_ref, fq_ref):
    z = z_ref[...]
    _, cs = _chain_residual(z, [q1_ref[...], q2_ref[...], q3_ref[...]])
    fq_ref[...] = (cs[0] + cs[1]) + cs[2]


def _tc_final(z, q1, q2, q3):
    row_spec = pl.BlockSpec((_BN, _D), lambda i: (i, 0))
    return pl.pallas_call(
        _tc_final_body,
        grid=(_NB,),
        in_specs=[row_spec] * 4,
        out_specs=row_spec,
        out_shape=jax.ShapeDtypeStruct((_N, _D), jnp.float32),
    )(z, q1, q2, q3)


_sc_mesh = plsc.VectorSubcoreMesh(core_axis_name="c", subcore_axis_name="s")


@functools.partial(
    pl.kernel,
    out_type=jax.ShapeDtypeStruct((_N, _D), jnp.float32),
    mesh=_sc_mesh,
    scratch_types=[
        pltpu.VMEM((_NCH, _CH), jnp.int32),
        pltpu.VMEM((_CH, _D), jnp.float32),
        pltpu.VMEM((_CH, _D), jnp.float32),
        pltpu.SemaphoreType.DMA,
        pltpu.SemaphoreType.DMA,
        pltpu.SemaphoreType.DMA,
        pltpu.SemaphoreType.DMA,
    ],
)
def _sc_gather(table_hbm, idx_hbm, out_hbm, idx_v, rows_a, rows_b,
               sem_ga, sem_gb, sem_sa, sem_sb):
    wid = lax.axis_index("s") * _NC + lax.axis_index("c")
    base = wid * _BPW
    bufs = (rows_a, rows_b)
    gsems = (sem_ga, sem_gb)
    ssems = (sem_sa, sem_sb)
    for c in range(_NCH):
        pltpu.sync_copy(idx_hbm.at[pl.ds(base + c * _CH, _CH)], idx_v.at[c])
    gathers = [None] * _NCH
    stores = [None] * _NCH
    for c in range(min(2, _NCH)):
        gathers[c] = pltpu.async_copy(
            table_hbm.at[idx_v.at[c]], bufs[c % 2], gsems[c % 2])
    for c in range(_NCH):
        gathers[c].wait()
        stores[c] = pltpu.async_copy(
            bufs[c % 2], out_hbm.at[pl.ds(base + c * _CH, _CH)], ssems[c % 2])
        if c + 2 < _NCH:
            stores[c].wait()
            gathers[c + 2] = pltpu.async_copy(
                table_hbm.at[idx_v.at[c + 2]], bufs[c % 2], gsems[c % 2])
    for c in range(max(0, _NCH - 2), _NCH):
        stores[c].wait()


def kernel(z, codebooks):
    w1 = codebooks[0]
    w2 = codebooks[1]
    w3 = codebooks[2]

    idx1, s1 = _tc_stage(z, (), w1)
    q1 = _sc_gather(w1, idx1)
    idx2, s2 = _tc_stage(z, (q1,), w2)
    q2 = _sc_gather(w2, idx2)
    idx3, s3 = _tc_stage(z, (q1, q2), w3)
    q3 = _sc_gather(w3, idx3)
    fq = _tc_final(z, q1, q2, q3)

    total = ((s1 + s2) + s3) / jnp.float32(_N * _D)
    return fq, total, total + 0.0


# speedup vs baseline: 1.1501x; 1.1501x over previous
import functools

import jax
import jax.numpy as jnp
from jax import lax
from jax.experimental import pallas as pl
from jax.experimental.pallas import tpu as pltpu
from jax.experimental.pallas import tpu_sc as plsc

_N, _D, _K = 16384, 256, 1024
_BN = 512
_NB = _N // _BN
_NC, _NS = 2, 16
_NW = _NC * _NS
_BPW = _N // _NW
_CH = 128
_NCH = _BPW // _CH


def _rne_bf16(x):
    u = lax.bitcast_convert_type(x, jnp.int32)
    r = (u + jnp.int32(0x7FFF) + ((u >> 16) & jnp.int32(1))) & jnp.int32(-65536)
    return lax.bitcast_convert_type(r, jnp.float32)


def _chain_residual(z, qs):
    r = z
    cs = []
    for q in qs:
        qr = _rne_bf16(q)
        c = r + (qr - r)
        cs.append(c)
        r = r - c
    return r, cs


def _tc_stage_body(nprev, *refs):
    z_ref = refs[0]
    q_refs = refs[1:1 + nprev]
    w_ref = refs[1 + nprev]
    idx_ref = refs[2 + nprev]
    loss_ref = refs[3 + nprev]
    wb_scr = refs[4 + nprev]
    c2_scr = refs[5 + nprev]

    i = pl.program_id(0)

    @pl.when(i == 0)
    def _():
        w = w_ref[...]
        wb_scr[...] = w.astype(jnp.bfloat16)
        c2 = jnp.sum(w * w, axis=1)
        c2_scr[...] = jnp.broadcast_to(c2[None, :], (8, _K))

    z = z_ref[...]
    r, _ = _chain_residual(z, [q[...] for q in q_refs])

    cl2 = lax.dot_general(
        (r + r).astype(jnp.bfloat16), wb_scr[...],
        (((1,), (1,)), ((), ())), preferred_element_type=jnp.float32)
    l2 = jnp.sum(r * r, axis=1, keepdims=True)
    d = (l2 - cl2) + c2_scr[0:1, :]

    m = jnp.min(d, axis=1, keepdims=True)
    cols = lax.broadcasted_iota(jnp.int32, d.shape, 1)
    idx = jnp.min(jnp.where(d == m, cols, jnp.int32(_K)), axis=1)
    idx_ref[...] = idx.reshape(1, 1, _BN)

    rows8 = lax.broadcasted_iota(jnp.int32, (8, 128), 0)
    cols8 = lax.broadcasted_iota(jnp.int32, (8, 128), 1)
    part = jnp.where((rows8 == 0) & (cols8 == 0), jnp.sum(m), 0.0)

    @pl.when(i == 0)
    def _():
        loss_ref[...] = jnp.zeros_like(loss_ref)

    loss_ref[...] += part


def _tc_stage(z, qs, w):
    nprev = len(qs)
    grid = (_NB,)
    row_spec = pl.BlockSpec((_BN, _D), lambda i: (i, 0))
    in_specs = ([row_spec] + [row_spec] * nprev
                + [pl.BlockSpec((_K, _D), lambda i: (0, 0))])
    out_specs = [
        pl.BlockSpec((1, 1, _BN), lambda i: (i, 0, 0)),
        pl.BlockSpec((8, 128), lambda i: (0, 0)),
    ]
    out_shape = [
        jax.ShapeDtypeStruct((_NB, 1, _BN), jnp.int32),
        jax.ShapeDtypeStruct((8, 128), jnp.float32),
    ]
    idx, loss = pl.pallas_call(
        functools.partial(_tc_stage_body, nprev),
        grid=grid,
        in_specs=in_specs,
        out_specs=out_specs,
        out_shape=out_shape,
        scratch_shapes=[
            pltpu.VMEM((_K, _D), jnp.bfloat16),
            pltpu.VMEM((8, _K), jnp.float32),
        ],
    )(z, *qs, w)
    return idx.reshape(_N), loss[0, 0]


def _tc_final_body(z_ref, q1_ref, q2_ref, q3_ref, fq_ref):
    z = z_ref[...]
    _, cs = _chain_residual(z, [q1_ref[...], q2_ref[...], q3_ref[...]])
    fq_ref[...] = (cs[0] + cs[1]) + cs[2]


def _tc_final(z, q1, q2, q3):
    row_spec = pl.BlockSpec((_BN, _D), lambda i: (i, 0))
    return pl.pallas_call(
        _tc_final_body,
        grid=(_NB,),
        in_specs=[row_spec] * 4,
        out_specs=row_spec,
        out_shape=jax.ShapeDtypeStruct((_N, _D), jnp.float32),
    )(z, q1, q2, q3)


_sc_mesh = plsc.VectorSubcoreMesh(core_axis_name="c", subcore_axis_name="s")


@functools.partial(
    pl.kernel,
    out_type=jax.ShapeDtypeStruct((_N, _D), jnp.float32),
    mesh=_sc_mesh,
    scratch_types=[
        pltpu.VMEM((_NCH, _CH), jnp.int32),
        pltpu.VMEM((_CH, _D), jnp.float32),
        pltpu.SemaphoreType.DMA,
    ],
)
def _sc_gather(table_hbm, idx_hbm, out_hbm, idx_v, rows_v, sem):
    wid = lax.axis_index("s") * _NC + lax.axis_index("c")
    base = wid * _BPW
    for c in range(_NCH):
        pltpu.sync_copy(idx_hbm.at[pl.ds(base + c * _CH, _CH)], idx_v.at[c])
        pltpu.async_copy(table_hbm.at[idx_v.at[c]], rows_v, sem).wait()
        pltpu.sync_copy(rows_v, out_hbm.at[pl.ds(base + c * _CH, _CH)])


def kernel(z, codebooks):
    w1 = codebooks[0]
    w2 = codebooks[1]
    w3 = codebooks[2]

    idx1, s1 = _tc_stage(z, (), w1)
    q1 = _sc_gather(w1, idx1)
    idx2, s2 = _tc_stage(z, (q1,), w2)
    q2 = _sc_gather(w2, idx2)
    idx3, s3 = _tc_stage(z, (q1, q2), w3)
    q3 = _sc_gather(w3, idx3)
    fq = _tc_final(z, q1, q2, q3)

    total = ((s1 + s2) + s3) / jnp.float32(_N * _D)
    return fq, total, total + 0.0
